# baseline (device time: 32457 ns/iter reference)
import jax
import jax.numpy as jnp
from jax import lax
from jax.experimental import pallas as pl
from jax.experimental.pallas import tpu as pltpu

N_DEV = 8
B, SQ, HQ, DH = 2, 512, 8, 64
SKV = 512
DM = HQ * DH
DMODEL = 768
WINDOW = 128
SCALE = 0.125

RB = 128
CB = 384
NCHUNK = 8

SCHEDULE = [(0, 0), (0, 1), (0, 2), (1, 0), (1, 1), (1, 2), (0, 3), (1, 3)]
CHUNK_ORDER = [4 * b + rb for b, rb in SCHEDULE]

STAGE1 = (1, 3, 4)
FWD = {1: (2, 5), 3: (7, 6)}
BARRIER_EDGES = [(0, 1), (0, 3), (0, 4), (1, 2), (1, 5), (3, 7), (3, 6)]


def kernel(x, Wq, K_ext, V_ext, Wo):
    K2 = K_ext.reshape(B, SKV, DM)
    V2 = V_ext.reshape(B, SKV, DM)

    def body(x_ref, wq_ref, k_ref, v_ref, wo_ref, out_ref,
             kv_peer, kv_snd, ctx_ref,
             kv_send_sem, kv_recv_sem, ctx_send, ctx_recv):
        pos = lax.axis_index("i")
        bf = jnp.bfloat16

        barrier = pltpu.get_barrier_semaphore()
        for a, bb in BARRIER_EDGES:
            @pl.when(pos == a)
            def _(bb=bb):
                pl.semaphore_signal(barrier, inc=1, device_id=(bb,),
                                    device_id_type=pl.DeviceIdType.MESH)

            @pl.when(pos == bb)
            def _(a=a):
                pl.semaphore_signal(barrier, inc=1, device_id=(a,),
                                    device_id_type=pl.DeviceIdType.MESH)
        deg = jnp.where((pos == 0) | (pos == 1) | (pos == 3), 3, 1)
        pl.semaphore_wait(barrier, deg)

        def ctx_copy(c, tgt, sem):
            return pltpu.make_async_remote_copy(
                src_ref=ctx_ref.at[c], dst_ref=ctx_ref.at[c],
                send_sem=sem, recv_sem=ctx_recv.at[c],
                device_id=(tgt,), device_id_type=pl.DeviceIdType.MESH)

        kv_rdma = pltpu.make_async_remote_copy(
            src_ref=kv_snd, dst_ref=kv_peer,
            send_sem=kv_send_sem, recv_sem=kv_recv_sem,
            device_id=(0,), device_id_type=pl.DeviceIdType.MESH)

        wo = wo_ref[...].astype(bf)

        def out_chunk(c):
            b, rb = c // 4, c % 4
            out_ref[b, rb * RB:(rb + 1) * RB] = jnp.dot(
                ctx_ref[c], wo, preferred_element_type=jnp.float32,
            ).astype(bf)

        @pl.when(pos == 1)
        def _():
            for b in range(B):
                kv_snd[0, b] = k_ref[b, :RB].astype(bf)
                kv_snd[1, b] = v_ref[b, :RB].astype(bf)
            kv_rdma.start()

        @pl.when(pos == 0)
        def _():
            wq = wq_ref[...].astype(bf)
            qs = [jnp.dot(x_ref[b].astype(bf), wq,
                          preferred_element_type=jnp.float32)
                  for b in range(B)]
            kbs = [k_ref[b].astype(bf) for b in range(B)]
            vbs = [v_ref[b].astype(bf) for b in range(B)]
            bands = {}
            for rb in range(4):
                r0, c0 = RB * rb, max(0, RB * rb - WINDOW)
                qi = r0 + lax.broadcasted_iota(jnp.int32, (RB, CB), 0)
                kj = c0 + lax.broadcasted_iota(jnp.int32, (RB, CB), 1)
                bands[rb] = (jnp.abs(qi - kj) <= WINDOW).astype(bf)
            waited = [False]
            for b, rb in SCHEDULE:
                if rb == 3 and not waited[0]:
                    kv_rdma.wait_recv()
                    waited[0] = True
                r0, c0 = RB * rb, max(0, RB * rb - WINDOW)
                if rb < 3:
                    kblk = kbs[b][c0:c0 + CB]
                    vblk = vbs[b][c0:c0 + CB]
                else:
                    kblk = jnp.concatenate(
                        [kbs[b][c0:SKV], kv_peer[0, b]], axis=0)
                    vblk = jnp.concatenate(
                        [vbs[b][c0:SKV], kv_peer[1, b]], axis=0)
                band = bands[rb]
                heads = []
                for h in range(HQ):
                    sl = slice(h * DH, (h + 1) * DH)
                    qh = qs[b][r0:r0 + RB, sl].astype(bf)
                    s = lax.dot_general(
                        qh, kblk[:, sl], (((1,), (1,)), ((), ())),
                        preferred_element_type=jnp.float32) * SCALE
                    w = jnp.exp(s.astype(bf)) * band
                    den = jnp.sum(w.astype(jnp.float32), axis=-1,
                                  keepdims=True)
                    c = lax.dot_general(
                        w, vblk[:, sl], (((1,), (0,)), ((), ())),
                        preferred_element_type=jnp.float32)
                    heads.append((c / den).astype(bf))
                chunk = 4 * b + rb
                ctx_ref[chunk] = jnp.concatenate(heads, axis=1)
                for t, tgt in enumerate(STAGE1):
                    ctx_copy(chunk, tgt, ctx_send.at[t, chunk]).start()
                out_chunk(chunk)

        @pl.when(pos != 0)
        def _():
            for c in CHUNK_ORDER:
                ctx_copy(c, 0, ctx_send.at[0, c]).wait_recv()
                for p, tgts in FWD.items():
                    @pl.when(pos == p)
                    def _(c=c, tgts=tgts):
                        for t, tgt in enumerate(tgts):
                            ctx_copy(c, tgt, ctx_send.at[t, c]).start()
                out_chunk(c)

        @pl.when(pos == 0)
        def _():
            for c in range(NCHUNK):
                for t, tgt in enumerate(STAGE1):
                    ctx_copy(c, tgt, ctx_send.at[t, c]).wait_send()

        @pl.when(pos == 1)
        def _():
            kv_rdma.wait_send()

        for p, tgts in FWD.items():
            @pl.when(pos == p)
            def _(tgts=tgts):
                for c in range(NCHUNK):
                    for t, tgt in enumerate(tgts):
                        ctx_copy(c, tgt, ctx_send.at[t, c]).wait_send()

    return pl.pallas_call(
        body,
        out_shape=jax.ShapeDtypeStruct((B, SQ, DMODEL), jnp.bfloat16),
        in_specs=[pl.BlockSpec(memory_space=pltpu.VMEM)] * 5,
        out_specs=pl.BlockSpec(memory_space=pltpu.VMEM),
        scratch_shapes=[
            pltpu.VMEM((2, B, RB, DM), jnp.bfloat16),
            pltpu.VMEM((2, B, RB, DM), jnp.bfloat16),
            pltpu.VMEM((NCHUNK, RB, DM), jnp.bfloat16),
            pltpu.SemaphoreType.DMA,
            pltpu.SemaphoreType.DMA,
            pltpu.SemaphoreType.DMA((3, NCHUNK)),
            pltpu.SemaphoreType.DMA((NCHUNK,)),
        ],
        compiler_params=pltpu.CompilerParams(
            vmem_limit_bytes=96 * 1024 * 1024,
            collective_id=0,
        ),
    )(x, Wq, K2, V2, Wo)
